# baseline (speedup 1.0000x reference)
import functools

import jax
import jax.numpy as jnp
from jax import lax
from jax.experimental import pallas as pl
from jax.experimental.pallas import tpu as pltpu
from jax.experimental.pallas import tpu_sc as plsc

_RADIUS = 2
_WIN = 2 * _RADIUS + 1
_ROWS = 4


def _mean_body(B, C, idx_ref, fields_ref, out_ref, win_ref, sem):
    def window_copy(bb):
        py = idx_ref[bb, 0]
        px = idx_ref[bb, 1]
        return pltpu.make_async_copy(
            fields_ref.at[
                bb, pl.ds(py - _RADIUS, _WIN), pl.ds(px - _RADIUS, _WIN), :
            ],
            win_ref.at[bb],
            sem,
        )

    for bb in range(B):
        window_copy(bb).start()
    for bb in range(B):
        window_copy(bb).wait()
        m = jnp.sum(win_ref[bb, :, :, 2:], axis=(0, 1)) * (1.0 / (_WIN * _WIN))
        out_ref[bb] = jnp.broadcast_to(
            m[None, None, :], out_ref.shape[1:]
        )


def _sc_broadcast_body(H, tmpl_ref, out_ref, rep, sem):
    b = lax.axis_index("s")
    half = lax.axis_index("c")

    pltpu.sync_copy(tmpl_ref.at[b], rep)

    rows_half = H // 2
    nchunk = rows_half // _ROWS
    base = half * rows_half
    copies = [
        pltpu.async_copy(
            rep, out_ref.at[b, pl.ds(base + _ROWS * k, _ROWS), :, :], sem
        )
        for k in range(nchunk)
    ]
    for cp in copies:
        cp.wait()


def kernel(fields, pump_indices):
    B, H, W, C = fields.shape
    Cout = C - 2
    idx = pump_indices.astype(jnp.int32)

    mean_grid = pltpu.PrefetchScalarGridSpec(
        num_scalar_prefetch=1,
        grid=(1,),
        in_specs=[pl.BlockSpec(memory_space=pl.ANY)],
        out_specs=pl.BlockSpec((B, _ROWS, W, Cout), lambda i, idx_ref: (0, 0, 0, 0)),
        scratch_shapes=[
            pltpu.VMEM((B, _WIN, _WIN, C), jnp.float32),
            pltpu.SemaphoreType.DMA,
        ],
    )
    tmpl = pl.pallas_call(
        functools.partial(_mean_body, B, C),
        grid_spec=mean_grid,
        out_shape=jax.ShapeDtypeStruct((B, _ROWS, W, Cout), jnp.float32),
    )(idx, fields)

    mesh = plsc.VectorSubcoreMesh(core_axis_name="c", subcore_axis_name="s")
    sc_fn = pl.kernel(
        functools.partial(_sc_broadcast_body, H),
        out_type=jax.ShapeDtypeStruct((B, H, W, Cout), jnp.float32),
        mesh=mesh,
        scratch_types=[
            pltpu.VMEM((_ROWS, W, Cout), jnp.float32),
            pltpu.SemaphoreType.DMA,
        ],
        compiler_params=pltpu.CompilerParams(use_tc_tiling_on_sc=True),
    )
    return sc_fn(tmpl)


# speedup vs baseline: 2.4092x; 2.4092x over previous
import jax
import jax.numpy as jnp
from jax import lax
from jax.experimental import pallas as pl
from jax.experimental.pallas import tpu as pltpu

_RADIUS = 2
_WIN = 2 * _RADIUS + 1
_BAND = 16
_TILE_H = 8


def _make_body(B, H, W, C):
    Cout = C - 2
    nchunk = H // _TILE_H

    def _body(idx_ref, fields_ref, out_ref, win_ref, tmpl_ref, wsem, osem):
        b = pl.program_id(0)
        nb = pl.num_programs(0)
        par = lax.rem(b, 2)

        def band_start(bb):
            py = idx_ref[bb, 0]
            ry0 = jnp.minimum((py - _RADIUS) // 8 * 8, H - _BAND)
            ry0 = pl.multiple_of(ry0, 8)
            return ry0

        def band_copy(bb, pp):
            return pltpu.make_async_copy(
                fields_ref.at[bb, pl.ds(2, Cout), pl.ds(band_start(bb), _BAND), :],
                win_ref.at[pp],
                wsem,
            )

        def chunk_copy(bb, pp, c):
            return pltpu.make_async_copy(
                tmpl_ref.at[pp],
                out_ref.at[bb, :, pl.ds(c * _TILE_H, _TILE_H), :],
                osem.at[pp],
            )

        @pl.when(b == 0)
        def _():
            band_copy(0, 0).start()

        @pl.when(b + 1 < nb)
        def _():
            band_copy(b + 1, 1 - par).start()

        @pl.when(b >= 2)
        def _():
            for c in range(nchunk):
                chunk_copy(b - 2, par, c).wait()

        band_copy(b, par).wait()

        py = idx_ref[b, 0]
        px = idx_ref[b, 1]
        dy = py - _RADIUS - band_start(b)
        cx = px - _RADIUS
        ri = lax.broadcasted_iota(jnp.int32, (_BAND, W), 0)
        ci = lax.broadcasted_iota(jnp.int32, (_BAND, W), 1)
        mask = (
            (ri >= dy) & (ri < dy + _WIN) & (ci >= cx) & (ci < cx + _WIN)
        ).astype(jnp.float32)
        m = jnp.sum(win_ref[par] * mask[None, :, :], axis=(1, 2), keepdims=True)
        m = m * (1.0 / (_WIN * _WIN))

        tmpl_ref[par] = jnp.broadcast_to(m, (Cout, _TILE_H, W))

        for c in range(nchunk):
            chunk_copy(b, par, c).start()

        @pl.when(b == nb - 1)
        def _():
            for c in range(nchunk):
                chunk_copy(b - 1, 1 - par, c).wait()
            for c in range(nchunk):
                chunk_copy(b, par, c).wait()

    return _body


def kernel(fields, pump_indices):
    B, H, W, C = fields.shape
    Cout = C - 2
    idx = pump_indices.astype(jnp.int32)
    fields_t = jnp.transpose(fields, (0, 3, 1, 2))

    grid_spec = pltpu.PrefetchScalarGridSpec(
        num_scalar_prefetch=1,
        grid=(B,),
        in_specs=[pl.BlockSpec(memory_space=pl.ANY)],
        out_specs=pl.BlockSpec(memory_space=pl.ANY),
        scratch_shapes=[
            pltpu.VMEM((2, Cout, _BAND, W), jnp.float32),
            pltpu.VMEM((2, Cout, _TILE_H, W), jnp.float32),
            pltpu.SemaphoreType.DMA,
            pltpu.SemaphoreType.DMA((2,)),
        ],
    )
    out_t = pl.pallas_call(
        _make_body(B, H, W, C),
        grid_spec=grid_spec,
        out_shape=jax.ShapeDtypeStruct((B, Cout, H, W), jnp.float32),
    )(idx, fields_t)
    return jnp.transpose(out_t, (0, 2, 3, 1))


# speedup vs baseline: 8.4774x; 3.5187x over previous
import jax
import jax.numpy as jnp
from jax import lax
from jax.experimental import pallas as pl
from jax.experimental.pallas import tpu as pltpu

_RADIUS = 2
_WIN = 2 * _RADIUS + 1
_TILE_H = 8


def _make_body(B, H, W, C):
    Cout = C - 2
    nchunk = H // _TILE_H

    def _body(idx_ref, fields_ref, out_ref, win_ref, tmpl_ref, wsem, osem):
        b = pl.program_id(0)
        nb = pl.num_programs(0)
        par = lax.rem(b, 2)

        def band_copy(bb, pp):
            py = idx_ref[bb, 0]
            return pltpu.make_async_copy(
                fields_ref.at[bb, pl.ds(py - _RADIUS, _WIN), :, :],
                win_ref.at[pp],
                wsem,
            )

        def chunk_copy(bb, pp, c):
            return pltpu.make_async_copy(
                tmpl_ref.at[pp],
                out_ref.at[bb, :, pl.ds(c * _TILE_H, _TILE_H), :],
                osem.at[pp],
            )

        @pl.when(b == 0)
        def _():
            band_copy(0, 0).start()

        @pl.when(b + 1 < nb)
        def _():
            band_copy(b + 1, 1 - par).start()

        @pl.when(b >= 2)
        def _():
            for c in range(nchunk):
                chunk_copy(b - 2, par, c).wait()

        band_copy(b, par).wait()

        px = idx_ref[b, 1]
        cx = px - _RADIUS
        ci = lax.iota(jnp.int32, W)
        cmask = ((ci >= cx) & (ci < cx + _WIN)).astype(jnp.float32)
        s1 = jnp.sum(win_ref[par], axis=0)
        s2 = jnp.sum(s1 * cmask[None, :], axis=1, keepdims=True)
        m = s2[2:, :] * (1.0 / (_WIN * _WIN))

        tmpl_ref[par] = jnp.broadcast_to(m[:, :, None], (Cout, _TILE_H, W))

        for c in range(nchunk):
            chunk_copy(b, par, c).start()

        @pl.when(b == nb - 1)
        def _():
            for c in range(nchunk):
                chunk_copy(b - 1, 1 - par, c).wait()
            for c in range(nchunk):
                chunk_copy(b, par, c).wait()

    return _body


def kernel(fields, pump_indices):
    B, H, W, C = fields.shape
    Cout = C - 2
    idx = pump_indices.astype(jnp.int32)
    fields_v = jnp.transpose(fields, (0, 1, 3, 2))

    grid_spec = pltpu.PrefetchScalarGridSpec(
        num_scalar_prefetch=1,
        grid=(B,),
        in_specs=[pl.BlockSpec(memory_space=pl.ANY)],
        out_specs=pl.BlockSpec(memory_space=pl.ANY),
        scratch_shapes=[
            pltpu.VMEM((2, _WIN, C, W), jnp.float32),
            pltpu.VMEM((2, Cout, _TILE_H, W), jnp.float32),
            pltpu.SemaphoreType.DMA,
            pltpu.SemaphoreType.DMA((2,)),
        ],
    )
    out_t = pl.pallas_call(
        _make_body(B, H, W, C),
        grid_spec=grid_spec,
        out_shape=jax.ShapeDtypeStruct((B, Cout, H, W), jnp.float32),
    )(idx, fields_v)
    return jnp.transpose(out_t, (0, 2, 3, 1))
